# baseline (speedup 1.0000x reference)
import functools

import jax
import jax.numpy as jnp
import numpy as np
from jax import lax
from jax.experimental import pallas as pl
from jax.experimental.pallas import tpu as pltpu
from jax.experimental.pallas import tpu_sc as plsc

_N = 50000
_D = 256
_H = 128
_G = 512
_BLK = 2000
_NBLK = _N // _BLK

_NC = 2
_NS = 16
_NW = _NC * _NS
_C = 80
_CPW = 19
_XTRA = 625 - _NW * _CPW
_MAXCH = _CPW + 1
_NV = _D // 16
_GRP = _C // 16

_BASES = [_C * (_CPW * w + min(w, _XTRA)) for w in range(_NW)]


def _score_body(x_ref, W1_ref, b1_ref, W2_ref, w_ref, z_ref, zacc_ref):
    i = pl.program_id(0)

    @pl.when(i == 0)
    def _init():
        zacc_ref[0] = 0.0

    x = x_ref[:]
    h = jnp.tanh(
        jax.lax.dot_general(x, W1_ref[:], (((1,), (0,)), ((), ())),
                            preferred_element_type=jnp.float32)
        + b1_ref[:])
    s = jnp.sum(h * W2_ref[:], axis=1, keepdims=True)
    e = jnp.exp(s)
    zacc_ref[0] += jnp.sum(e)
    w_ref[:] = x * e
    z_ref[0] = zacc_ref[0]


_mesh = plsc.VectorSubcoreMesh(core_axis_name="c", subcore_axis_name="s",
                               num_cores=_NC, num_subcores=_NS)


@functools.partial(
    pl.kernel,
    out_type=[
        jax.ShapeDtypeStruct((_G, _D), jnp.float32),
        jax.ShapeDtypeStruct((_NW, _D), jnp.float32),
    ],
    mesh=_mesh,
    scratch_types=[
        pltpu.VMEM((_C, _D), jnp.float32),
        pltpu.VMEM((_C, _D), jnp.float32),
        pltpu.VMEM((_C + 16,), jnp.int32),
        pltpu.VMEM((_C + 16,), jnp.int32),
        pltpu.VMEM((16,), jnp.int32),
        pltpu.VMEM((_D,), jnp.float32),
        pltpu.VMEM((_D,), jnp.float32),
        pltpu.VMEM((_D,), jnp.float32),
        pltpu.VMEM((_D,), jnp.float32),
        pltpu.SemaphoreType.DMA,
        pltpu.SemaphoreType.DMA,
        pltpu.SemaphoreType.DMA,
        pltpu.SemaphoreType.DMA,
    ],
)
def _seg_kernel(w_hbm, batch_hbm, out_hbm, side_hbm,
                wv0, wv1, iv0, iv1, pv, tb, bb, rb, zb, ws0, ws1, is0, is1):
    cid = lax.axis_index("c")
    sid = lax.axis_index("s")
    wid = cid * _NS + sid
    r0 = (_CPW * wid + jnp.minimum(wid, _XTRA)) * _C
    nch = _CPW + jnp.where(wid < _XTRA, 1, 0)

    zero16 = jnp.zeros((16,), jnp.float32)
    for v in range(_NV):
        zb[pl.ds(16 * v, 16)] = zero16
        bb[pl.ds(16 * v, 16)] = zero16
        tb[pl.ds(16 * v, 16)] = zero16

    @pl.when(wid > 0)
    def _loadprev():
        pltpu.sync_copy(batch_hbm.at[pl.ds(r0 - 16, 16)], pv)

    prev_id = jnp.where(wid > 0, pv[:][15], jnp.int32(-1))

    def copies(k, wbuf, ibuf, wsem, isem):
        base = r0 + k * _C
        hw = pltpu.make_async_copy(w_hbm.at[pl.ds(base, _C), :], wbuf, wsem)
        hi = pltpu.make_async_copy(batch_hbm.at[pl.ds(base, _C)],
                                   ibuf.at[pl.ds(0, _C)], isem)
        return hw, hi

    def start(k, wbuf, ibuf, wsem, isem):
        hw, hi = copies(k, wbuf, ibuf, wsem, isem)

        @pl.when(k < nch)
        def _():
            hw.start()
            hi.start()

    def wait(k, wbuf, ibuf, wsem, isem):
        hw, hi = copies(k, wbuf, ibuf, wsem, isem)

        @pl.when(k < nch)
        def _():
            hw.wait()
            hi.wait()

    def zfill(lo, hi):
        def zrow(g, c):
            pltpu.sync_copy(zb, out_hbm.at[g])
            return c
        lax.fori_loop(lo + 1, hi, zrow, 0)

    def flush(cur):
        for v in range(_NV):
            sl = pl.ds(16 * v, 16)
            t = tb[sl]
            rb[sl] = t - bb[sl]
            bb[sl] = t

        def to_side():
            pltpu.sync_copy(rb, side_hbm.at[wid])

        def to_out():
            pltpu.sync_copy(rb, out_hbm.at[cur])

        lax.cond(cur == prev_id, to_side, to_out)

    def do_rows(wv, iv, active, cur):
        for j in range(_GRP):
            base = 16 * j
            idv = iv[pl.ds(base, 16)]
            last = idv[15]
            fast = last == cur

            @pl.when(fast & active)
            def _fast():
                for v in range(_NV):
                    sl = pl.ds(16 * v, 16)
                    terms = [wv[base + r, sl] for r in range(16)]
                    while len(terms) > 1:
                        terms = [a + b for a, b in
                                 zip(terms[::2], terms[1::2])]
                    tb[sl] = tb[sl] + terms[0]

            @pl.when(jnp.logical_not(fast) & active)
            def _slow():
                def srow(r, c):
                    idr = iv[pl.ds(base + r, 16)][0]
                    ch = idr != c

                    @pl.when(ch)
                    def _boundary():
                        flush(c)
                        zfill(c, idr)

                    for v in range(_NV):
                        sl = pl.ds(16 * v, 16)
                        tb[sl] = tb[sl] + wv[base + r, sl]
                    return idr

                lax.fori_loop(0, 16, srow, cur)

            cur = jnp.where(active, last, cur)
        return cur

    start(0, wv0, iv0, ws0, is0)
    start(1, wv1, iv1, ws1, is1)
    cur = prev_id

    def pair_body(p, cur):
        k0 = 2 * p
        k1 = 2 * p + 1
        wait(k0, wv0, iv0, ws0, is0)
        cur = do_rows(wv0, iv0, k0 < nch, cur)
        start(k0 + 2, wv0, iv0, ws0, is0)
        wait(k1, wv1, iv1, ws1, is1)
        cur = do_rows(wv1, iv1, k1 < nch, cur)
        start(k1 + 2, wv1, iv1, ws1, is1)
        return cur

    cur = lax.fori_loop(0, _MAXCH // 2, pair_body, cur)

    flush(cur)

    @pl.when(wid == _NW - 1)
    def _endfill():
        zfill(cur, _G)


def _combine_body(z_ref, sid_ref, main_ref, side_ref, out_ref):
    gids = jax.lax.broadcasted_iota(jnp.int32, (_G, 1), 0)
    oh = (sid_ref[:] == gids).astype(jnp.float32)
    out_ref[:] = (main_ref[:]
                  + jax.lax.dot_general(oh, side_ref[:],
                                        (((1,), (0,)), ((), ())),
                                        preferred_element_type=jnp.float32)
                  ) * (1.0 / z_ref[0])


def kernel(x, batch, W1, b1, W2, b2):
    batch_i = batch.astype(jnp.int32)
    b1r = b1.reshape(1, _H)
    W2r = W2.reshape(1, _H)

    weighted, z = pl.pallas_call(
        _score_body,
        grid=(_NBLK,),
        in_specs=[
            pl.BlockSpec((_BLK, _D), lambda i: (i, 0)),
            pl.BlockSpec((_D, _H), lambda i: (0, 0)),
            pl.BlockSpec((1, _H), lambda i: (0, 0)),
            pl.BlockSpec((1, _H), lambda i: (0, 0)),
        ],
        out_specs=[
            pl.BlockSpec((_BLK, _D), lambda i: (i, 0)),
            pl.BlockSpec(memory_space=pltpu.SMEM),
        ],
        out_shape=[
            jax.ShapeDtypeStruct((_N, _D), jnp.float32),
            jax.ShapeDtypeStruct((1,), jnp.float32),
        ],
        scratch_shapes=[pltpu.SMEM((1,), jnp.float32)],
        compiler_params=pltpu.CompilerParams(
            dimension_semantics=("arbitrary",)),
    )(x, W1, b1r, W2r)

    main, side = _seg_kernel(weighted, batch_i)

    side_ids = batch_i[np.asarray(_BASES)].reshape(1, _NW)
    out = pl.pallas_call(
        _combine_body,
        in_specs=[
            pl.BlockSpec(memory_space=pltpu.SMEM),
            pl.BlockSpec((1, _NW), lambda: (0, 0)),
            pl.BlockSpec((_G, _D), lambda: (0, 0)),
            pl.BlockSpec((_NW, _D), lambda: (0, 0)),
        ],
        out_specs=pl.BlockSpec((_G, _D), lambda: (0, 0)),
        out_shape=jax.ShapeDtypeStruct((_G, _D), jnp.float32),
    )(z, side_ids, main, side)
    return out


# speedup vs baseline: 1.5677x; 1.5677x over previous
import functools

import jax
import jax.numpy as jnp
import numpy as np
from jax import lax
from jax.experimental import pallas as pl
from jax.experimental.pallas import tpu as pltpu
from jax.experimental.pallas import tpu_sc as plsc

_N = 50000
_D = 256
_H = 128
_G = 512
_BLK = 2000
_NBLK = _N // _BLK

_NC = 2
_NS = 16
_NW = _NC * _NS
_C = 80
_CPW = 19
_XTRA = 625 - _NW * _CPW
_MAXCH = _CPW + 1
_NV = _D // 16
_GRP = _C // 16

_BASES = [_C * (_CPW * w + min(w, _XTRA)) for w in range(_NW)]


def _score_body(x_ref, W1_ref, b1_ref, W2_ref, w_ref, z_ref, zacc_ref):
    i = pl.program_id(0)

    @pl.when(i == 0)
    def _init():
        zacc_ref[0] = 0.0

    x = x_ref[:]
    h = jnp.tanh(
        jax.lax.dot_general(x, W1_ref[:], (((1,), (0,)), ((), ())),
                            preferred_element_type=jnp.float32)
        + b1_ref[:])
    s = jnp.sum(h * W2_ref[:], axis=1, keepdims=True)
    e = jnp.exp(s)
    zacc_ref[0] += jnp.sum(e)
    w_ref[:] = x * e
    z_ref[0] = zacc_ref[0]


_mesh = plsc.VectorSubcoreMesh(core_axis_name="c", subcore_axis_name="s",
                               num_cores=_NC, num_subcores=_NS)


@functools.partial(
    pl.kernel,
    out_type=[
        jax.ShapeDtypeStruct((_G, _D), jnp.float32),
        jax.ShapeDtypeStruct((_NW, _D), jnp.float32),
    ],
    mesh=_mesh,
    scratch_types=[
        pltpu.VMEM((_C, _D), jnp.float32),
        pltpu.VMEM((_C, _D), jnp.float32),
        pltpu.VMEM((_C + 16,), jnp.int32),
        pltpu.VMEM((_C + 16,), jnp.int32),
        pltpu.VMEM((16,), jnp.int32),
        pltpu.VMEM((_D,), jnp.float32),
        pltpu.VMEM((_D,), jnp.float32),
        pltpu.VMEM((_D,), jnp.float32),
        pltpu.VMEM((_D,), jnp.float32),
        pltpu.SemaphoreType.DMA,
        pltpu.SemaphoreType.DMA,
        pltpu.SemaphoreType.DMA,
        pltpu.SemaphoreType.DMA,
    ],
)
def _seg_kernel(w_hbm, batch_hbm, out_hbm, side_hbm,
                wv0, wv1, iv0, iv1, pv, tb, bb, rb, zb, ws0, ws1, is0, is1):
    cid = lax.axis_index("c")
    sid = lax.axis_index("s")
    wid = cid * _NS + sid
    r0 = (_CPW * wid + jnp.minimum(wid, _XTRA)) * _C
    nch = _CPW + jnp.where(wid < _XTRA, 1, 0)

    zero16 = jnp.zeros((16,), jnp.float32)
    for v in range(_NV):
        zb[pl.ds(16 * v, 16)] = zero16
        bb[pl.ds(16 * v, 16)] = zero16
        tb[pl.ds(16 * v, 16)] = zero16

    @pl.when(wid > 0)
    def _loadprev():
        pltpu.sync_copy(batch_hbm.at[pl.ds(r0 - 16, 16)], pv)

    prev_id = jnp.where(wid > 0, pv[:][15], jnp.int32(-1))

    def copies(k, wbuf, ibuf, wsem, isem):
        base = r0 + k * _C
        hw = pltpu.make_async_copy(w_hbm.at[pl.ds(base, _C), :], wbuf, wsem)
        hi = pltpu.make_async_copy(batch_hbm.at[pl.ds(base, _C)],
                                   ibuf.at[pl.ds(0, _C)], isem)
        return hw, hi

    def start(k, wbuf, ibuf, wsem, isem):
        hw, hi = copies(k, wbuf, ibuf, wsem, isem)

        @pl.when(k < nch)
        def _():
            hw.start()
            hi.start()

    def wait(k, wbuf, ibuf, wsem, isem):
        hw, hi = copies(k, wbuf, ibuf, wsem, isem)

        @pl.when(k < nch)
        def _():
            hw.wait()
            hi.wait()

    def zfill(lo, hi):
        def zrow(g, c):
            pltpu.sync_copy(zb, out_hbm.at[g])
            return c
        lax.fori_loop(lo + 1, hi, zrow, 0)

    def flush(cur):
        for v in range(_NV):
            sl = pl.ds(16 * v, 16)
            t = tb[sl]
            rb[sl] = t - bb[sl]
            bb[sl] = t

        def to_side():
            pltpu.sync_copy(rb, side_hbm.at[wid])

        def to_out():
            pltpu.sync_copy(rb, out_hbm.at[cur])

        lax.cond(cur == prev_id, to_side, to_out)

    def do_rows(wv, iv, active, cur):
        def group(j, cur):
            base = 16 * j
            idv = iv[pl.ds(base, 16)]
            last = idv[15]
            fast = last == cur

            @pl.when(fast & active)
            def _fast():
                for v in range(_NV):
                    sl = pl.ds(16 * v, 16)
                    terms = [wv[base + r, sl] for r in range(16)]
                    while len(terms) > 1:
                        terms = [a + b for a, b in
                                 zip(terms[::2], terms[1::2])]
                    tb[sl] = tb[sl] + terms[0]

            @pl.when(jnp.logical_not(fast) & active)
            def _slow():
                def srow(r, c):
                    idr = iv[pl.ds(base + r, 16)][0]
                    ch = idr != c

                    @pl.when(ch)
                    def _boundary():
                        flush(c)
                        zfill(c, idr)

                    for v in range(_NV):
                        sl = pl.ds(16 * v, 16)
                        tb[sl] = tb[sl] + wv[base + r, sl]
                    return idr

                lax.fori_loop(0, 16, srow, cur)

            return jnp.where(active, last, cur)

        return lax.fori_loop(0, _GRP, group, cur)

    start(0, wv0, iv0, ws0, is0)
    start(1, wv1, iv1, ws1, is1)
    cur = prev_id

    def pair_body(p, cur):
        k0 = 2 * p
        k1 = 2 * p + 1
        wait(k0, wv0, iv0, ws0, is0)
        cur = do_rows(wv0, iv0, k0 < nch, cur)
        start(k0 + 2, wv0, iv0, ws0, is0)
        wait(k1, wv1, iv1, ws1, is1)
        cur = do_rows(wv1, iv1, k1 < nch, cur)
        start(k1 + 2, wv1, iv1, ws1, is1)
        return cur

    cur = lax.fori_loop(0, _MAXCH // 2, pair_body, cur)

    flush(cur)

    @pl.when(wid == _NW - 1)
    def _endfill():
        zfill(cur, _G)


def _combine_body(z_ref, sid_ref, main_ref, side_ref, out_ref):
    gids = jax.lax.broadcasted_iota(jnp.int32, (_G, 1), 0)
    oh = (sid_ref[:] == gids).astype(jnp.float32)
    out_ref[:] = (main_ref[:]
                  + jax.lax.dot_general(oh, side_ref[:],
                                        (((1,), (0,)), ((), ())),
                                        preferred_element_type=jnp.float32)
                  ) * (1.0 / z_ref[0])


def kernel(x, batch, W1, b1, W2, b2):
    batch_i = batch.astype(jnp.int32)
    b1r = b1.reshape(1, _H)
    W2r = W2.reshape(1, _H)

    weighted, z = pl.pallas_call(
        _score_body,
        grid=(_NBLK,),
        in_specs=[
            pl.BlockSpec((_BLK, _D), lambda i: (i, 0)),
            pl.BlockSpec((_D, _H), lambda i: (0, 0)),
            pl.BlockSpec((1, _H), lambda i: (0, 0)),
            pl.BlockSpec((1, _H), lambda i: (0, 0)),
        ],
        out_specs=[
            pl.BlockSpec((_BLK, _D), lambda i: (i, 0)),
            pl.BlockSpec(memory_space=pltpu.SMEM),
        ],
        out_shape=[
            jax.ShapeDtypeStruct((_N, _D), jnp.float32),
            jax.ShapeDtypeStruct((1,), jnp.float32),
        ],
        scratch_shapes=[pltpu.SMEM((1,), jnp.float32)],
        compiler_params=pltpu.CompilerParams(
            dimension_semantics=("arbitrary",)),
    )(x, W1, b1r, W2r)

    main, side = _seg_kernel(weighted, batch_i)

    side_ids = batch_i[np.asarray(_BASES)].reshape(1, _NW)
    out = pl.pallas_call(
        _combine_body,
        in_specs=[
            pl.BlockSpec(memory_space=pltpu.SMEM),
            pl.BlockSpec((1, _NW), lambda: (0, 0)),
            pl.BlockSpec((_G, _D), lambda: (0, 0)),
            pl.BlockSpec((_NW, _D), lambda: (0, 0)),
        ],
        out_specs=pl.BlockSpec((_G, _D), lambda: (0, 0)),
        out_shape=jax.ShapeDtypeStruct((_G, _D), jnp.float32),
    )(z, side_ids, main, side)
    return out
